# baseline (speedup 1.0000x reference)
import functools

import jax
import jax.numpy as jnp
from jax.experimental import pallas as pl
from jax.experimental.pallas import tpu as pltpu

DIM = 128
LN_EPS = 1e-5
AGG_EPS = 1e-5
ROW_TILE = 1024


def _cdiv(a, b):
    return -(-a // b)


def _round_up(n, m):
    return _cdiv(n, m) * m


def _tile(n):
    if n >= ROW_TILE:
        return ROW_TILE
    return max(8, _round_up(n, 8))


def _pad_rows(x, p):
    n = x.shape[0]
    if n == p:
        return x
    return jnp.pad(x, [(0, p - n)] + [(0, 0)] * (x.ndim - 1))


def _ln(h, g, b):
    mu = jnp.mean(h, axis=-1, keepdims=True)
    var = jnp.mean(jnp.square(h - mu), axis=-1, keepdims=True)
    return (h - mu) * jax.lax.rsqrt(var + LN_EPS) * g + b


def _silu(x):
    return x * jax.nn.sigmoid(x)


def _row_spec(t, d):
    return pl.BlockSpec((t, d), lambda i: (i, 0))


def _const_spec(shape):
    return pl.BlockSpec(shape, lambda i: (0, 0))


def _node_proj_kernel(x_ref, w_ref, b_ref, hs_ref, pd_ref, ps_ref):
    h = jnp.dot(x_ref[...], w_ref[...],
                preferred_element_type=jnp.float32) + b_ref[...]
    hs_ref[...] = h[:, :DIM]
    ps_ref[...] = h[:, DIM:3 * DIM]
    a = h[:, 3 * DIM:]
    pd_ref[:, :DIM] = a
    pd_ref[:, DIM:] = a


def _edge_kernel(e_ref, gd_ref, gs_ref, wc_ref, bs_ref, g_ref, bt_ref,
                 eo_ref, ms_ref, *, n_valid, masked):
    e = e_ref[...]
    gs = gs_ref[...]
    e_hat = (jnp.dot(e, wc_ref[...], preferred_element_type=jnp.float32)
             + gd_ref[...] + gs[:, DIM:] + bs_ref[...])
    eo_ref[...] = e + _silu(_ln(e_hat, g_ref[...], bt_ref[...]))
    sig = jax.nn.sigmoid(e_hat)
    if masked:
        t = e_ref.shape[0]
        row = pl.program_id(0) * t + jax.lax.broadcasted_iota(
            jnp.int32, (t, 1), 0)
        sig = sig * (row < n_valid).astype(jnp.float32)
    ms_ref[:, :DIM] = sig * gs[:, :DIM]
    ms_ref[:, DIM:] = sig


def _edge_gather_kernel(dst_ref, src_ref, e_ref, pd_ref, ps_ref, wc_ref,
                        bs_ref, g_ref, bt_ref, eo_ref, ms_ref, gd_s, gs_s,
                        *, n_valid, masked):
    te = e_ref.shape[0]
    for u in range(te):
        d = dst_ref[0, 0, u]
        s = src_ref[0, 0, u]
        dc = pd_ref[pl.ds(pl.multiple_of((d >> 3) << 3, 8), 8), :]
        sc = ps_ref[pl.ds(pl.multiple_of((s >> 3) << 3, 8), 8), :]
        sub = u & 7
        dr = pltpu.roll(dc, (u - d) & 7, axis=0)
        sr = pltpu.roll(sc, (u - s) & 7, axis=0)
        gd_s[u:u + 1, :] = dr[sub:sub + 1, :]
        gs_s[u:u + 1, :] = sr[sub:sub + 1, :]

    e = e_ref[...]
    gs = gs_s[...]
    e_hat = (jnp.dot(e, wc_ref[...], preferred_element_type=jnp.float32)
             + gd_s[...] + gs[:, DIM:] + bs_ref[...])
    eo_ref[...] = e + _silu(_ln(e_hat, g_ref[...], bt_ref[...]))
    sig = jax.nn.sigmoid(e_hat)
    if masked:
        row = pl.program_id(0) * te + jax.lax.broadcasted_iota(
            jnp.int32, (te, 1), 0)
        sig = sig * (row < n_valid).astype(jnp.float32)
    ms_ref[:, :DIM] = sig * gs[:, :DIM]
    ms_ref[:, DIM:] = sig


def _node_upd_kernel(x_ref, hs_ref, agg_ref, g_ref, bt_ref, o_ref):
    agg32 = agg_ref[...]
    agg = agg32[:, :DIM] / (agg32[:, DIM:] + AGG_EPS)
    h = hs_ref[...] + agg
    o_ref[...] = x_ref[...] + _silu(_ln(h, g_ref[...], bt_ref[...]))


def _layer_pre(x, src, dst, p, fused):
    n = x.shape[0]
    n_edge = src.shape[0]
    tn = _tile(n)
    pn = _round_up(n, tn)
    te = _tile(n_edge)
    pe = _round_up(n_edge, te)

    w_cat = jnp.concatenate(
        [p["w_self"], p["w_nbr"], p["w_B"], p["w_A"]], axis=1)
    zb = jnp.zeros_like(p["b_self"])
    b_cat = jnp.concatenate([p["b_self"], p["b_nbr"], zb, zb], axis=1)

    hs, pd, ps = pl.pallas_call(
        _node_proj_kernel,
        out_shape=(jax.ShapeDtypeStruct((pn, DIM), jnp.float32),
                   jax.ShapeDtypeStruct((pn, 2 * DIM), jnp.float32),
                   jax.ShapeDtypeStruct((pn, 2 * DIM), jnp.float32)),
        grid=(pn // tn,),
        in_specs=[_row_spec(tn, DIM),
                  _const_spec((DIM, 4 * DIM)), _const_spec((1, 4 * DIM))],
        out_specs=(_row_spec(tn, DIM), _row_spec(tn, 2 * DIM),
                   _row_spec(tn, 2 * DIM)),
        compiler_params=pltpu.CompilerParams(
            dimension_semantics=("parallel",)),
    )(_pad_rows(x, pn), w_cat, b_cat)

    if fused:
        return hs, pd, ps

    gd = _pad_rows(pd.at[dst].get(mode="promise_in_bounds"), pe)
    gs = _pad_rows(ps.at[src].get(mode="promise_in_bounds"), pe)
    return hs, gd, gs


def _layer_post(x, e, hs, gd, gs, src, dst, p, fused):
    n, n_edge = x.shape[0], e.shape[0]
    tn = _tile(n)
    pn = _round_up(n, tn)
    te = _tile(n_edge)
    pe = _round_up(n_edge, te)

    b_sum = p["b_A"] + p["b_B"] + p["b_C"]

    if fused:
        tef = min(256, te)
        idx3 = [jnp.pad(v, (0, pe - n_edge)).reshape(pe // tef, 1, tef)
                for v in (dst, src)]
        smem_spec = pl.BlockSpec((1, 1, tef), lambda i: (i, 0, 0),
                                 memory_space=pltpu.SMEM)
        e_new, ms = pl.pallas_call(
            functools.partial(_edge_gather_kernel, n_valid=n_edge,
                              masked=(pe != n_edge)),
            out_shape=(jax.ShapeDtypeStruct((pe, DIM), jnp.float32),
                       jax.ShapeDtypeStruct((pe, 2 * DIM), jnp.float32)),
            grid=(pe // tef,),
            in_specs=[smem_spec, smem_spec, _row_spec(tef, DIM),
                      pl.BlockSpec((pn, DIM), lambda i: (0, 0)),
                      pl.BlockSpec((pn, 2 * DIM), lambda i: (0, 0)),
                      _const_spec((DIM, DIM)), _const_spec((1, DIM)),
                      _const_spec((1, DIM)), _const_spec((1, DIM))],
            out_specs=(_row_spec(tef, DIM), _row_spec(tef, 2 * DIM)),
            scratch_shapes=[pltpu.VMEM((tef, DIM), jnp.float32),
                            pltpu.VMEM((tef, 2 * DIM), jnp.float32)],
            compiler_params=pltpu.CompilerParams(
                dimension_semantics=("parallel",)),
        )(idx3[0], idx3[1], _pad_rows(e, pe), gd, gs,
          p["w_C"], b_sum, p["ln_e_g"], p["ln_e_b"])
    else:
        e_new, ms = pl.pallas_call(
            functools.partial(_edge_kernel, n_valid=n_edge,
                              masked=(pe != n_edge)),
            out_shape=(jax.ShapeDtypeStruct((pe, DIM), jnp.float32),
                       jax.ShapeDtypeStruct((pe, 2 * DIM), jnp.float32)),
            grid=(pe // te,),
            in_specs=[_row_spec(te, DIM), _row_spec(te, DIM),
                      _row_spec(te, 2 * DIM),
                      _const_spec((DIM, DIM)), _const_spec((1, DIM)),
                      _const_spec((1, DIM)), _const_spec((1, DIM))],
            out_specs=(_row_spec(te, DIM), _row_spec(te, 2 * DIM)),
            compiler_params=pltpu.CompilerParams(
                dimension_semantics=("parallel",)),
        )(_pad_rows(e, pe), gd, gs,
          p["w_C"], b_sum, p["ln_e_g"], p["ln_e_b"])
    e_new = e_new[:n_edge]

    agg = jax.ops.segment_sum(
        ms[:n_edge], dst, num_segments=n,
        mode=jax.lax.GatherScatterMode.PROMISE_IN_BOUNDS)

    x_new = pl.pallas_call(
        _node_upd_kernel,
        out_shape=jax.ShapeDtypeStruct((pn, DIM), jnp.float32),
        grid=(pn // tn,),
        in_specs=[_row_spec(tn, DIM), _row_spec(tn, DIM),
                  _row_spec(tn, 2 * DIM),
                  _const_spec((1, DIM)), _const_spec((1, DIM))],
        out_specs=_row_spec(tn, DIM),
        compiler_params=pltpu.CompilerParams(
            dimension_semantics=("parallel",)),
    )(_pad_rows(x, pn), hs, _pad_rows(agg, pn),
      p["ln_x_g"], p["ln_x_b"])
    return x_new[:n], e_new


def kernel(h_atm, h_bnd, h_ang, edge_index_G, edge_index_A, bnd_ang_0_w_self, bnd_ang_0_b_self, bnd_ang_0_w_nbr, bnd_ang_0_b_nbr, bnd_ang_0_w_A, bnd_ang_0_b_A, bnd_ang_0_w_B, bnd_ang_0_b_B, bnd_ang_0_w_C, bnd_ang_0_b_C, bnd_ang_0_ln_x_g, bnd_ang_0_ln_x_b, bnd_ang_0_ln_e_g, bnd_ang_0_ln_e_b, bnd_ang_1_w_self, bnd_ang_1_b_self, bnd_ang_1_w_nbr, bnd_ang_1_b_nbr, bnd_ang_1_w_A, bnd_ang_1_b_A, bnd_ang_1_w_B, bnd_ang_1_b_B, bnd_ang_1_w_C, bnd_ang_1_b_C, bnd_ang_1_ln_x_g, bnd_ang_1_ln_x_b, bnd_ang_1_ln_e_g, bnd_ang_1_ln_e_b, atm_bnd_0_w_self, atm_bnd_0_b_self, atm_bnd_0_w_nbr, atm_bnd_0_b_nbr, atm_bnd_0_w_A, atm_bnd_0_b_A, atm_bnd_0_w_B, atm_bnd_0_b_B, atm_bnd_0_w_C, atm_bnd_0_b_C, atm_bnd_0_ln_x_g, atm_bnd_0_ln_x_b, atm_bnd_0_ln_e_g, atm_bnd_0_ln_e_b, atm_bnd_1_w_self, atm_bnd_1_b_self, atm_bnd_1_w_nbr, atm_bnd_1_b_nbr, atm_bnd_1_w_A, atm_bnd_1_b_A, atm_bnd_1_w_B, atm_bnd_1_b_B, atm_bnd_1_w_C, atm_bnd_1_b_C, atm_bnd_1_ln_x_g, atm_bnd_1_ln_x_b, atm_bnd_1_ln_e_g, atm_bnd_1_ln_e_b):
    def _p(w_self, b_self, w_nbr, b_nbr, w_A, b_A, w_B, b_B, w_C, b_C,
           ln_x_g, ln_x_b, ln_e_g, ln_e_b):
        return dict(w_self=w_self, b_self=b_self, w_nbr=w_nbr, b_nbr=b_nbr,
                    w_A=w_A, b_A=b_A, w_B=w_B, b_B=b_B, w_C=w_C, b_C=b_C,
                    ln_x_g=ln_x_g, ln_x_b=ln_x_b, ln_e_g=ln_e_g, ln_e_b=ln_e_b)

    params = {
        "bnd_ang": [
            _p(bnd_ang_0_w_self, bnd_ang_0_b_self, bnd_ang_0_w_nbr, bnd_ang_0_b_nbr,
               bnd_ang_0_w_A, bnd_ang_0_b_A, bnd_ang_0_w_B, bnd_ang_0_b_B,
               bnd_ang_0_w_C, bnd_ang_0_b_C, bnd_ang_0_ln_x_g, bnd_ang_0_ln_x_b,
               bnd_ang_0_ln_e_g, bnd_ang_0_ln_e_b),
            _p(bnd_ang_1_w_self, bnd_ang_1_b_self, bnd_ang_1_w_nbr, bnd_ang_1_b_nbr,
               bnd_ang_1_w_A, bnd_ang_1_b_A, bnd_ang_1_w_B, bnd_ang_1_b_B,
               bnd_ang_1_w_C, bnd_ang_1_b_C, bnd_ang_1_ln_x_g, bnd_ang_1_ln_x_b,
               bnd_ang_1_ln_e_g, bnd_ang_1_ln_e_b),
        ],
        "atm_bnd": [
            _p(atm_bnd_0_w_self, atm_bnd_0_b_self, atm_bnd_0_w_nbr, atm_bnd_0_b_nbr,
               atm_bnd_0_w_A, atm_bnd_0_b_A, atm_bnd_0_w_B, atm_bnd_0_b_B,
               atm_bnd_0_w_C, atm_bnd_0_b_C, atm_bnd_0_ln_x_g, atm_bnd_0_ln_x_b,
               atm_bnd_0_ln_e_g, atm_bnd_0_ln_e_b),
            _p(atm_bnd_1_w_self, atm_bnd_1_b_self, atm_bnd_1_w_nbr, atm_bnd_1_b_nbr,
               atm_bnd_1_w_A, atm_bnd_1_b_A, atm_bnd_1_w_B, atm_bnd_1_b_B,
               atm_bnd_1_w_C, atm_bnd_1_b_C, atm_bnd_1_ln_x_g, atm_bnd_1_ln_x_b,
               atm_bnd_1_ln_e_g, atm_bnd_1_ln_e_b),
        ],
    }

    src_A, dst_A = edge_index_A[0], edge_index_A[1]
    src_G, dst_G = edge_index_G[0], edge_index_G[1]
    for i in range(2):
        bnd_pre = _layer_pre(h_bnd, src_A, dst_A, params["bnd_ang"][i],
                             fused=False)
        atm_pre = _layer_pre(h_atm, src_G, dst_G, params["atm_bnd"][i],
                             fused=True)
        h_bnd, h_ang = _layer_post(h_bnd, h_ang, *bnd_pre, src_A, dst_A,
                                   params["bnd_ang"][i], fused=False)
        h_atm, h_bnd = _layer_post(h_atm, h_bnd, *atm_pre, src_G, dst_G,
                                   params["atm_bnd"][i], fused=True)

    return {"h_atm": h_atm, "h_bnd": h_bnd, "h_ang": h_ang,
            "edge_index_G": edge_index_G, "edge_index_A": edge_index_A}


# speedup vs baseline: 1.1786x; 1.1786x over previous
import functools

import jax
import jax.numpy as jnp
from jax.experimental import pallas as pl
from jax.experimental.pallas import tpu as pltpu

DIM = 128
LN_EPS = 1e-5
AGG_EPS = 1e-5
ROW_TILE = 1024


def _cdiv(a, b):
    return -(-a // b)


def _round_up(n, m):
    return _cdiv(n, m) * m


def _tile(n):
    if n >= ROW_TILE:
        return ROW_TILE
    return max(8, _round_up(n, 8))


def _pad_rows(x, p):
    n = x.shape[0]
    if n == p:
        return x
    return jnp.pad(x, [(0, p - n)] + [(0, 0)] * (x.ndim - 1))


def _ln(h, g, b):
    mu = jnp.mean(h, axis=-1, keepdims=True)
    var = jnp.mean(jnp.square(h - mu), axis=-1, keepdims=True)
    return (h - mu) * jax.lax.rsqrt(var + LN_EPS) * g + b


def _silu(x):
    return x * jax.nn.sigmoid(x)


def _row_spec(t, d):
    return pl.BlockSpec((t, d), lambda i: (i, 0))


def _const_spec(shape):
    return pl.BlockSpec(shape, lambda i: (0, 0))


def _node_proj_kernel(x_ref, w_ref, b_ref, hs_ref, pd_ref, ps_ref):
    h = jnp.dot(x_ref[...], w_ref[...],
                preferred_element_type=jnp.float32) + b_ref[...]
    hs_ref[...] = h[:, :DIM]
    ps_ref[...] = h[:, DIM:3 * DIM]
    a = h[:, 3 * DIM:]
    if pd_ref.shape[1] == 2 * DIM:
        pd_ref[:, :DIM] = a
        pd_ref[:, DIM:] = a
    else:
        pd_ref[...] = a


def _edge_kernel(e_ref, gd_ref, gs_ref, wc_ref, bs_ref, g_ref, bt_ref,
                 eo_ref, ms_ref, *, n_valid, masked):
    e = e_ref[...]
    gs = gs_ref[...]
    e_hat = (jnp.dot(e, wc_ref[...], preferred_element_type=jnp.float32)
             + gd_ref[...] + gs[:, DIM:] + bs_ref[...])
    eo_ref[...] = e + _silu(_ln(e_hat, g_ref[...], bt_ref[...]))
    sig = jax.nn.sigmoid(e_hat)
    if masked:
        t = e_ref.shape[0]
        row = pl.program_id(0) * t + jax.lax.broadcasted_iota(
            jnp.int32, (t, 1), 0)
        sig = sig * (row < n_valid).astype(jnp.float32)
    ms_ref[:, :DIM] = sig * gs[:, :DIM]
    ms_ref[:, DIM:] = sig


def _node_upd_kernel(x_ref, hs_ref, agg_ref, g_ref, bt_ref, o_ref):
    agg32 = agg_ref[...]
    agg = agg32[:, :DIM] / (agg32[:, DIM:] + AGG_EPS)
    h = hs_ref[...] + agg
    o_ref[...] = x_ref[...] + _silu(_ln(h, g_ref[...], bt_ref[...]))


def _layer_pre(x, src, dst, p, wide_gd):
    n = x.shape[0]
    n_edge = src.shape[0]
    tn = _tile(n)
    pn = _round_up(n, tn)
    te = _tile(n_edge)
    pe = _round_up(n_edge, te)

    w_cat = jnp.concatenate(
        [p["w_self"], p["w_nbr"], p["w_B"], p["w_A"]], axis=1)
    zb = jnp.zeros_like(p["b_self"])
    b_cat = jnp.concatenate([p["b_self"], p["b_nbr"], zb, zb], axis=1)

    pdw = 2 * DIM if wide_gd else DIM
    hs, pd, ps = pl.pallas_call(
        _node_proj_kernel,
        out_shape=(jax.ShapeDtypeStruct((pn, DIM), jnp.float32),
                   jax.ShapeDtypeStruct((pn, pdw), jnp.float32),
                   jax.ShapeDtypeStruct((pn, 2 * DIM), jnp.float32)),
        grid=(pn // tn,),
        in_specs=[_row_spec(tn, DIM),
                  _const_spec((DIM, 4 * DIM)), _const_spec((1, 4 * DIM))],
        out_specs=(_row_spec(tn, DIM), _row_spec(tn, pdw),
                   _row_spec(tn, 2 * DIM)),
        compiler_params=pltpu.CompilerParams(
            dimension_semantics=("parallel",)),
    )(_pad_rows(x, pn), w_cat, b_cat)

    gd = _pad_rows(pd.at[dst].get(mode="promise_in_bounds"), pe)
    gs = _pad_rows(ps.at[src].get(mode="promise_in_bounds"), pe)
    return hs, gd, gs


def _layer_post(x, e, hs, gd, gs, src, dst, p):
    n, n_edge = x.shape[0], e.shape[0]
    tn = _tile(n)
    pn = _round_up(n, tn)
    te = _tile(n_edge)
    pe = _round_up(n_edge, te)

    b_sum = p["b_A"] + p["b_B"] + p["b_C"]

    e_new, ms = pl.pallas_call(
        functools.partial(_edge_kernel, n_valid=n_edge,
                          masked=(pe != n_edge)),
        out_shape=(jax.ShapeDtypeStruct((pe, DIM), jnp.float32),
                   jax.ShapeDtypeStruct((pe, 2 * DIM), jnp.float32)),
        grid=(pe // te,),
        in_specs=[_row_spec(te, DIM), _row_spec(te, DIM),
                  _row_spec(te, 2 * DIM),
                  _const_spec((DIM, DIM)), _const_spec((1, DIM)),
                  _const_spec((1, DIM)), _const_spec((1, DIM))],
        out_specs=(_row_spec(te, DIM), _row_spec(te, 2 * DIM)),
        compiler_params=pltpu.CompilerParams(
            dimension_semantics=("parallel",)),
    )(_pad_rows(e, pe), gd, gs,
      p["w_C"], b_sum, p["ln_e_g"], p["ln_e_b"])
    e_new = e_new[:n_edge]

    agg = jax.ops.segment_sum(
        ms[:n_edge], dst, num_segments=n,
        mode=jax.lax.GatherScatterMode.PROMISE_IN_BOUNDS)

    x_new = pl.pallas_call(
        _node_upd_kernel,
        out_shape=jax.ShapeDtypeStruct((pn, DIM), jnp.float32),
        grid=(pn // tn,),
        in_specs=[_row_spec(tn, DIM), _row_spec(tn, DIM),
                  _row_spec(tn, 2 * DIM),
                  _const_spec((1, DIM)), _const_spec((1, DIM))],
        out_specs=_row_spec(tn, DIM),
        compiler_params=pltpu.CompilerParams(
            dimension_semantics=("parallel",)),
    )(_pad_rows(x, pn), hs, _pad_rows(agg, pn),
      p["ln_x_g"], p["ln_x_b"])
    return x_new[:n], e_new


def kernel(h_atm, h_bnd, h_ang, edge_index_G, edge_index_A, bnd_ang_0_w_self, bnd_ang_0_b_self, bnd_ang_0_w_nbr, bnd_ang_0_b_nbr, bnd_ang_0_w_A, bnd_ang_0_b_A, bnd_ang_0_w_B, bnd_ang_0_b_B, bnd_ang_0_w_C, bnd_ang_0_b_C, bnd_ang_0_ln_x_g, bnd_ang_0_ln_x_b, bnd_ang_0_ln_e_g, bnd_ang_0_ln_e_b, bnd_ang_1_w_self, bnd_ang_1_b_self, bnd_ang_1_w_nbr, bnd_ang_1_b_nbr, bnd_ang_1_w_A, bnd_ang_1_b_A, bnd_ang_1_w_B, bnd_ang_1_b_B, bnd_ang_1_w_C, bnd_ang_1_b_C, bnd_ang_1_ln_x_g, bnd_ang_1_ln_x_b, bnd_ang_1_ln_e_g, bnd_ang_1_ln_e_b, atm_bnd_0_w_self, atm_bnd_0_b_self, atm_bnd_0_w_nbr, atm_bnd_0_b_nbr, atm_bnd_0_w_A, atm_bnd_0_b_A, atm_bnd_0_w_B, atm_bnd_0_b_B, atm_bnd_0_w_C, atm_bnd_0_b_C, atm_bnd_0_ln_x_g, atm_bnd_0_ln_x_b, atm_bnd_0_ln_e_g, atm_bnd_0_ln_e_b, atm_bnd_1_w_self, atm_bnd_1_b_self, atm_bnd_1_w_nbr, atm_bnd_1_b_nbr, atm_bnd_1_w_A, atm_bnd_1_b_A, atm_bnd_1_w_B, atm_bnd_1_b_B, atm_bnd_1_w_C, atm_bnd_1_b_C, atm_bnd_1_ln_x_g, atm_bnd_1_ln_x_b, atm_bnd_1_ln_e_g, atm_bnd_1_ln_e_b):
    def _p(w_self, b_self, w_nbr, b_nbr, w_A, b_A, w_B, b_B, w_C, b_C,
           ln_x_g, ln_x_b, ln_e_g, ln_e_b):
        return dict(w_self=w_self, b_self=b_self, w_nbr=w_nbr, b_nbr=b_nbr,
                    w_A=w_A, b_A=b_A, w_B=w_B, b_B=b_B, w_C=w_C, b_C=b_C,
                    ln_x_g=ln_x_g, ln_x_b=ln_x_b, ln_e_g=ln_e_g, ln_e_b=ln_e_b)

    params = {
        "bnd_ang": [
            _p(bnd_ang_0_w_self, bnd_ang_0_b_self, bnd_ang_0_w_nbr, bnd_ang_0_b_nbr,
               bnd_ang_0_w_A, bnd_ang_0_b_A, bnd_ang_0_w_B, bnd_ang_0_b_B,
               bnd_ang_0_w_C, bnd_ang_0_b_C, bnd_ang_0_ln_x_g, bnd_ang_0_ln_x_b,
               bnd_ang_0_ln_e_g, bnd_ang_0_ln_e_b),
            _p(bnd_ang_1_w_self, bnd_ang_1_b_self, bnd_ang_1_w_nbr, bnd_ang_1_b_nbr,
               bnd_ang_1_w_A, bnd_ang_1_b_A, bnd_ang_1_w_B, bnd_ang_1_b_B,
               bnd_ang_1_w_C, bnd_ang_1_b_C, bnd_ang_1_ln_x_g, bnd_ang_1_ln_x_b,
               bnd_ang_1_ln_e_g, bnd_ang_1_ln_e_b),
        ],
        "atm_bnd": [
            _p(atm_bnd_0_w_self, atm_bnd_0_b_self, atm_bnd_0_w_nbr, atm_bnd_0_b_nbr,
               atm_bnd_0_w_A, atm_bnd_0_b_A, atm_bnd_0_w_B, atm_bnd_0_b_B,
               atm_bnd_0_w_C, atm_bnd_0_b_C, atm_bnd_0_ln_x_g, atm_bnd_0_ln_x_b,
               atm_bnd_0_ln_e_g, atm_bnd_0_ln_e_b),
            _p(atm_bnd_1_w_self, atm_bnd_1_b_self, atm_bnd_1_w_nbr, atm_bnd_1_b_nbr,
               atm_bnd_1_w_A, atm_bnd_1_b_A, atm_bnd_1_w_B, atm_bnd_1_b_B,
               atm_bnd_1_w_C, atm_bnd_1_b_C, atm_bnd_1_ln_x_g, atm_bnd_1_ln_x_b,
               atm_bnd_1_ln_e_g, atm_bnd_1_ln_e_b),
        ],
    }

    src_A, dst_A = edge_index_A[0], edge_index_A[1]
    src_G, dst_G = edge_index_G[0], edge_index_G[1]
    for i in range(2):
        bnd_pre = _layer_pre(h_bnd, src_A, dst_A, params["bnd_ang"][i],
                             wide_gd=True)
        atm_pre = _layer_pre(h_atm, src_G, dst_G, params["atm_bnd"][i],
                             wide_gd=False)
        h_bnd, h_ang = _layer_post(h_bnd, h_ang, *bnd_pre, src_A, dst_A,
                                   params["bnd_ang"][i])
        h_atm, h_bnd = _layer_post(h_atm, h_bnd, *atm_pre, src_G, dst_G,
                                   params["atm_bnd"][i])

    return {"h_atm": h_atm, "h_bnd": h_bnd, "h_ang": h_ang,
            "edge_index_G": edge_index_G, "edge_index_A": edge_index_A}
